# baseline (speedup 1.0000x reference)
import functools

import jax
import jax.numpy as jnp
from jax import lax
from jax.experimental import pallas as pl
from jax.experimental.pallas import tpu as pltpu
from jax.experimental.pallas import tpu_sc as plsc

N = 10000
E = 320000
D = 128
DE = 16
L = 3

NC = 2
NS = 16
NW = NC * NS
EPW = E // NW
C = 40
NCHUNK = EPW // C
NFL = N // C
LANES = 16
G = D // LANES


def _agg_body(x_hbm, ee_hbm, idxr_hbm, out_hbm,
              idx_b, eb3, rb2, acc, se3, sg2, ss3, si4):
    c = lax.axis_index("c")
    s = lax.axis_index("s")
    wid = c * NS + s
    ebase = wid * EPW

    def zrow(r, carry):
        for j in range(G):
            eb3[0, r, pl.ds(j * LANES, LANES)] = jnp.zeros((LANES,),
                                                           jnp.float32)
        return carry
    lax.fori_loop(0, C, zrow, 0)
    for t in range((NFL + NS - 1) // NS):
        idx = s + NS * t

        @pl.when(idx < NFL)
        def _():
            pltpu.sync_copy(eb3.at[0], acc.at[pl.ds(idx * C, C)])
    plsc.subcore_barrier()

    def issue_idx(k, s4):
        pltpu.async_copy(idxr_hbm.at[wid, k], idx_b.at[s4], si4.at[s4])

    def wait_idx(k, s4):
        pltpu.make_async_copy(idxr_hbm.at[wid, k], idx_b.at[s4],
                              si4.at[s4]).wait()

    def issue_loads(k, s3, s2, s4):
        pltpu.async_copy(ee_hbm.at[pl.ds(ebase + k * C, C)], eb3.at[s3],
                         se3.at[s3])
        pltpu.async_copy(x_hbm.at[idx_b.at[s4, 0]], rb2.at[s2], sg2.at[s2])

    def wait_loads(k, s3, s2, s4):
        pltpu.make_async_copy(ee_hbm.at[pl.ds(ebase + k * C, C)], eb3.at[s3],
                              se3.at[s3]).wait()
        pltpu.make_async_copy(x_hbm.at[idx_b.at[s4, 0]], rb2.at[s2],
                              sg2.at[s2]).wait()

    def compute(s3, s2):
        def edge(e, carry):
            for j in range(G):
                sl = pl.ds(j * LANES, LANES)
                eb3[s3, e, sl] = jnp.maximum(
                    rb2[s2, e, sl] + eb3[s3, e, sl], 0.0)
            return carry
        lax.fori_loop(0, C, edge, 0, unroll=2)

    def issue_scatter(k, s3, s4):
        pltpu.async_copy(eb3.at[s3], acc.at[idx_b.at[s4, 1]], ss3.at[s3],
                         add=True)

    def wait_scatter(k, s3, s4):
        pltpu.make_async_copy(eb3.at[s3], acc.at[idx_b.at[s4, 1]],
                              ss3.at[s3]).wait()

    def step(k, off):
        s3, s2, s4 = off % 3, off % 2, off % 4
        if not isinstance(k, int) or k >= 2:
            wait_scatter(k - 2, (off - 2) % 3, (off - 2) % 4)
        if isinstance(k, int) and k + 1 >= NCHUNK:
            pass
        else:
            wait_idx(k + 1, (off + 1) % 4)
            issue_loads(k + 1, (off + 1) % 3, (off + 1) % 2, (off + 1) % 4)
        if not isinstance(k, int) or k + 2 < NCHUNK:
            issue_idx(k + 2, (off + 2) % 4)
        wait_loads(k, s3, s2, s4)
        compute(s3, s2)
        issue_scatter(k, s3, s4)

    issue_idx(0, 0)
    issue_idx(1, 1)
    wait_idx(0, 0)
    issue_loads(0, 0, 0, 0)
    for k in range(12):
        step(k, k)

    nblk = (NCHUNK - 12 - 10) // 12

    def block(i, carry):
        kb = 12 * i
        for off in range(12):
            step(kb + off, off)
        return carry
    lax.fori_loop(1, 1 + nblk, block, 0)

    for k in range(NCHUNK - 10, NCHUNK):
        step(k, k % 12)
    wait_scatter(NCHUNK - 2, (NCHUNK - 2) % 3, (NCHUNK - 2) % 4)
    wait_scatter(NCHUNK - 1, (NCHUNK - 1) % 3, (NCHUNK - 1) % 4)

    plsc.subcore_barrier()
    for t in range((NFL + NS - 1) // NS):
        idx = s + NS * t

        @pl.when(idx < NFL)
        def _():
            off = idx * C
            pltpu.sync_copy(acc.at[pl.ds(off, C)],
                            out_hbm.at[c, pl.ds(off, C)])


_agg = pl.kernel(
    _agg_body,
    out_type=jax.ShapeDtypeStruct((NC, N, D), jnp.float32),
    mesh=plsc.VectorSubcoreMesh(core_axis_name="c", subcore_axis_name="s",
                                num_cores=NC, num_subcores=NS),
    scratch_types=[
        pltpu.VMEM((4, 2, C), jnp.int32),
        pltpu.VMEM((3, C, D), jnp.float32),
        pltpu.VMEM((2, C, D), jnp.float32),
        pltpu.VMEM_SHARED((N, D), jnp.float32),
        pltpu.SemaphoreType.DMA((3,)),
        pltpu.SemaphoreType.DMA((2,)),
        pltpu.SemaphoreType.DMA((3,)),
        pltpu.SemaphoreType.DMA((4,)),
    ],
)


BE = 4000


def _ee_body(ea_ref, we_ref, be_ref, out_ref):
    r = jnp.dot(ea_ref[...], we_ref[0],
                preferred_element_type=jnp.float32) + be_ref[0]
    out_ref[...] = r[None]


_edge_proj = pl.pallas_call(
    _ee_body,
    grid=(L, E // BE),
    in_specs=[
        pl.BlockSpec((BE, DE), lambda l, e: (e, 0)),
        pl.BlockSpec((1, DE, D), lambda l, e: (l, 0, 0)),
        pl.BlockSpec((1, 1, D), lambda l, e: (l, 0, 0)),
    ],
    out_specs=pl.BlockSpec((1, BE, D), lambda l, e: (l, e, 0)),
    out_shape=jax.ShapeDtypeStruct((L, E, D), jnp.float32),
)


R = 2000


def _node_common(x_ref, a_ref, w1_ref, b1_ref, w2_ref, b2_ref, g_ref, bb_ref):
    x = x_ref[...]
    h = x + a_ref[0] + a_ref[1]
    t = jnp.maximum(
        jnp.dot(h, w1_ref[...], preferred_element_type=jnp.float32)
        + b1_ref[0], 0.0)
    t = jnp.dot(t, w2_ref[...], preferred_element_type=jnp.float32) + b2_ref[0]
    mu = jnp.mean(t, axis=-1, keepdims=True)
    var = jnp.mean((t - mu) ** 2, axis=-1, keepdims=True)
    t = (t - mu) / jnp.sqrt(var + 1e-5) * g_ref[0] + bb_ref[0]
    return jax.nn.gelu(t) + x


def _node_mid_body(x_ref, a_ref, w1_ref, b1_ref, w2_ref, b2_ref, g_ref,
                   bb_ref, out_ref):
    out_ref[...] = _node_common(x_ref, a_ref, w1_ref, b1_ref, w2_ref, b2_ref,
                                g_ref, bb_ref)


def _node_last_body(x_ref, a_ref, w1_ref, b1_ref, w2_ref, b2_ref, g_ref,
                    bb_ref, wo_ref, bo_ref, out_ref):
    y = _node_common(x_ref, a_ref, w1_ref, b1_ref, w2_ref, b2_ref,
                     g_ref, bb_ref)
    out_ref[...] = jnp.dot(y, wo_ref[...],
                           preferred_element_type=jnp.float32) + bo_ref[0]


_NODE_SPECS = [
    pl.BlockSpec((R, D), lambda i: (i, 0)),
    pl.BlockSpec((NC, R, D), lambda i: (0, i, 0)),
    pl.BlockSpec((D, D), lambda i: (0, 0)),
    pl.BlockSpec((1, D), lambda i: (0, 0)),
    pl.BlockSpec((D, D), lambda i: (0, 0)),
    pl.BlockSpec((1, D), lambda i: (0, 0)),
    pl.BlockSpec((1, D), lambda i: (0, 0)),
    pl.BlockSpec((1, D), lambda i: (0, 0)),
]

_node_mid = pl.pallas_call(
    _node_mid_body,
    grid=(N // R,),
    in_specs=_NODE_SPECS,
    out_specs=pl.BlockSpec((R, D), lambda i: (i, 0)),
    out_shape=jax.ShapeDtypeStruct((N, D), jnp.float32),
)

_node_last = pl.pallas_call(
    _node_last_body,
    grid=(N // R,),
    in_specs=_NODE_SPECS + [
        pl.BlockSpec((D, D), lambda i: (0, 0)),
        pl.BlockSpec((1, D), lambda i: (0, 0)),
    ],
    out_specs=pl.BlockSpec((R, D), lambda i: (i, 0)),
    out_shape=jax.ShapeDtypeStruct((N, D), jnp.float32),
)


def kernel(x, edge_index, edge_attr, We, be, W1, b1, W2, b2, ln_g, ln_b,
           Wout, bout):
    idxr = jnp.stack([edge_index[0].reshape(NW, NCHUNK, C),
                      edge_index[1].reshape(NW, NCHUNK, C)], axis=2)
    ee = _edge_proj(edge_attr, We, be[:, None])
    for i in range(L):
        agg2 = _agg(x, ee[i], idxr)
        args = (x, agg2, W1[i], b1[i][None], W2[i], b2[i][None],
                ln_g[i][None], ln_b[i][None])
        if i < L - 1:
            x = _node_mid(*args)
        else:
            x = _node_last(*args, Wout, bout[None])
    return x


# speedup vs baseline: 1.4650x; 1.4650x over previous
import functools

import jax
import jax.numpy as jnp
from jax import lax
from jax.experimental import pallas as pl
from jax.experimental.pallas import tpu as pltpu
from jax.experimental.pallas import tpu_sc as plsc

N = 10000
E = 320000
D = 128
DE = 16
L = 3

NC = 2
NS = 16
NW = NC * NS
EPW = E // NW
C = 40
NCHUNK = EPW // C
NFL = N // C
LANES = 16
G = D // LANES


def _agg_body(x_hbm, ee_hbm, idxr_hbm, out_hbm,
              idx_b, eb3, rb2, acc, se3, sg2, ss3, si4):
    c = lax.axis_index("c")
    s = lax.axis_index("s")
    wid = c * NS + s
    ebase = wid * EPW

    @plsc.parallel_loop(0, C, unroll=2)
    def zrow(r):
        for j in range(G):
            eb3[0, r, pl.ds(j * LANES, LANES)] = jnp.zeros((LANES,),
                                                           jnp.float32)
    for t in range((NFL + NS - 1) // NS):
        idx = s + NS * t

        @pl.when(idx < NFL)
        def _():
            pltpu.sync_copy(eb3.at[0], acc.at[pl.ds(idx * C, C)])
    plsc.subcore_barrier()

    def issue_idx(k, s4):
        pltpu.async_copy(idxr_hbm.at[wid, k], idx_b.at[s4], si4.at[s4])

    def wait_idx(k, s4):
        pltpu.make_async_copy(idxr_hbm.at[wid, k], idx_b.at[s4],
                              si4.at[s4]).wait()

    def issue_loads(k, s3, s2, s4):
        pltpu.async_copy(ee_hbm.at[pl.ds(ebase + k * C, C)], eb3.at[s3],
                         se3.at[s3])
        pltpu.async_copy(x_hbm.at[idx_b.at[s4, 0]], rb2.at[s2], sg2.at[s2])

    def wait_loads(k, s3, s2, s4):
        pltpu.make_async_copy(ee_hbm.at[pl.ds(ebase + k * C, C)], eb3.at[s3],
                              se3.at[s3]).wait()
        pltpu.make_async_copy(x_hbm.at[idx_b.at[s4, 0]], rb2.at[s2],
                              sg2.at[s2]).wait()

    def compute(s3, s2):
        @plsc.parallel_loop(0, C, unroll=4)
        def edge(e):
            for j in range(G):
                sl = pl.ds(j * LANES, LANES)
                eb3[s3, e, sl] = jnp.maximum(
                    rb2[s2, e, sl] + eb3[s3, e, sl], 0.0)

    def issue_scatter(k, s3, s4):
        pltpu.async_copy(eb3.at[s3], acc.at[idx_b.at[s4, 1]], ss3.at[s3],
                         add=True)

    def wait_scatter(k, s3, s4):
        pltpu.make_async_copy(eb3.at[s3], acc.at[idx_b.at[s4, 1]],
                              ss3.at[s3]).wait()

    def step(k, off):
        s3, s2, s4 = off % 3, off % 2, off % 4
        if not isinstance(k, int) or k >= 2:
            wait_scatter(k - 2, (off - 2) % 3, (off - 2) % 4)
        if isinstance(k, int) and k + 1 >= NCHUNK:
            pass
        else:
            wait_idx(k + 1, (off + 1) % 4)
            issue_loads(k + 1, (off + 1) % 3, (off + 1) % 2, (off + 1) % 4)
        if not isinstance(k, int) or k + 2 < NCHUNK:
            issue_idx(k + 2, (off + 2) % 4)
        wait_loads(k, s3, s2, s4)
        compute(s3, s2)
        issue_scatter(k, s3, s4)

    issue_idx(0, 0)
    issue_idx(1, 1)
    wait_idx(0, 0)
    issue_loads(0, 0, 0, 0)
    for k in range(12):
        step(k, k)

    nblk = (NCHUNK - 12 - 10) // 12

    def block(i, carry):
        kb = 12 * i
        for off in range(12):
            step(kb + off, off)
        return carry
    lax.fori_loop(1, 1 + nblk, block, 0)

    for k in range(NCHUNK - 10, NCHUNK):
        step(k, k % 12)
    wait_scatter(NCHUNK - 2, (NCHUNK - 2) % 3, (NCHUNK - 2) % 4)
    wait_scatter(NCHUNK - 1, (NCHUNK - 1) % 3, (NCHUNK - 1) % 4)

    plsc.subcore_barrier()
    for t in range((NFL + NS - 1) // NS):
        idx = s + NS * t

        @pl.when(idx < NFL)
        def _():
            off = idx * C
            pltpu.sync_copy(acc.at[pl.ds(off, C)],
                            out_hbm.at[c, pl.ds(off, C)])


_agg = pl.kernel(
    _agg_body,
    out_type=jax.ShapeDtypeStruct((NC, N, D), jnp.float32),
    mesh=plsc.VectorSubcoreMesh(core_axis_name="c", subcore_axis_name="s",
                                num_cores=NC, num_subcores=NS),
    scratch_types=[
        pltpu.VMEM((4, 2, C), jnp.int32),
        pltpu.VMEM((3, C, D), jnp.float32),
        pltpu.VMEM((2, C, D), jnp.float32),
        pltpu.VMEM_SHARED((N, D), jnp.float32),
        pltpu.SemaphoreType.DMA((3,)),
        pltpu.SemaphoreType.DMA((2,)),
        pltpu.SemaphoreType.DMA((3,)),
        pltpu.SemaphoreType.DMA((4,)),
    ],
)


BE = 4000


def _ee_body(ea_ref, we_ref, be_ref, out_ref):
    r = jnp.dot(ea_ref[...], we_ref[0],
                preferred_element_type=jnp.float32) + be_ref[0]
    out_ref[...] = r[None]


_edge_proj = pl.pallas_call(
    _ee_body,
    grid=(L, E // BE),
    in_specs=[
        pl.BlockSpec((BE, DE), lambda l, e: (e, 0)),
        pl.BlockSpec((1, DE, D), lambda l, e: (l, 0, 0)),
        pl.BlockSpec((1, 1, D), lambda l, e: (l, 0, 0)),
    ],
    out_specs=pl.BlockSpec((1, BE, D), lambda l, e: (l, e, 0)),
    out_shape=jax.ShapeDtypeStruct((L, E, D), jnp.float32),
)


R = 2000


def _node_common(x_ref, a_ref, w1_ref, b1_ref, w2_ref, b2_ref, g_ref, bb_ref):
    x = x_ref[...]
    h = x + a_ref[0] + a_ref[1]
    t = jnp.maximum(
        jnp.dot(h, w1_ref[...], preferred_element_type=jnp.float32)
        + b1_ref[0], 0.0)
    t = jnp.dot(t, w2_ref[...], preferred_element_type=jnp.float32) + b2_ref[0]
    mu = jnp.mean(t, axis=-1, keepdims=True)
    var = jnp.mean((t - mu) ** 2, axis=-1, keepdims=True)
    t = (t - mu) / jnp.sqrt(var + 1e-5) * g_ref[0] + bb_ref[0]
    return jax.nn.gelu(t) + x


def _node_mid_body(x_ref, a_ref, w1_ref, b1_ref, w2_ref, b2_ref, g_ref,
                   bb_ref, out_ref):
    out_ref[...] = _node_common(x_ref, a_ref, w1_ref, b1_ref, w2_ref, b2_ref,
                                g_ref, bb_ref)


def _node_last_body(x_ref, a_ref, w1_ref, b1_ref, w2_ref, b2_ref, g_ref,
                    bb_ref, wo_ref, bo_ref, out_ref):
    y = _node_common(x_ref, a_ref, w1_ref, b1_ref, w2_ref, b2_ref,
                     g_ref, bb_ref)
    out_ref[...] = jnp.dot(y, wo_ref[...],
                           preferred_element_type=jnp.float32) + bo_ref[0]


_NODE_SPECS = [
    pl.BlockSpec((R, D), lambda i: (i, 0)),
    pl.BlockSpec((NC, R, D), lambda i: (0, i, 0)),
    pl.BlockSpec((D, D), lambda i: (0, 0)),
    pl.BlockSpec((1, D), lambda i: (0, 0)),
    pl.BlockSpec((D, D), lambda i: (0, 0)),
    pl.BlockSpec((1, D), lambda i: (0, 0)),
    pl.BlockSpec((1, D), lambda i: (0, 0)),
    pl.BlockSpec((1, D), lambda i: (0, 0)),
]

_node_mid = pl.pallas_call(
    _node_mid_body,
    grid=(N // R,),
    in_specs=_NODE_SPECS,
    out_specs=pl.BlockSpec((R, D), lambda i: (i, 0)),
    out_shape=jax.ShapeDtypeStruct((N, D), jnp.float32),
)

_node_last = pl.pallas_call(
    _node_last_body,
    grid=(N // R,),
    in_specs=_NODE_SPECS + [
        pl.BlockSpec((D, D), lambda i: (0, 0)),
        pl.BlockSpec((1, D), lambda i: (0, 0)),
    ],
    out_specs=pl.BlockSpec((R, D), lambda i: (i, 0)),
    out_shape=jax.ShapeDtypeStruct((N, D), jnp.float32),
)


def kernel(x, edge_index, edge_attr, We, be, W1, b1, W2, b2, ln_g, ln_b,
           Wout, bout):
    idxr = jnp.stack([edge_index[0].reshape(NW, NCHUNK, C),
                      edge_index[1].reshape(NW, NCHUNK, C)], axis=2)
    ee = _edge_proj(edge_attr, We, be[:, None])
    for i in range(L):
        agg2 = _agg(x, ee[i], idxr)
        args = (x, agg2, W1[i], b1[i][None], W2[i], b2[i][None],
                ln_g[i][None], ln_b[i][None])
        if i < L - 1:
            x = _node_mid(*args)
        else:
            x = _node_last(*args, Wout, bout[None])
    return x


# speedup vs baseline: 2.0531x; 1.4014x over previous
import functools

import jax
import jax.numpy as jnp
from jax import lax
from jax.experimental import pallas as pl
from jax.experimental.pallas import tpu as pltpu
from jax.experimental.pallas import tpu_sc as plsc

N = 10000
E = 320000
D = 128
DE = 16
L = 3

NC = 2
NS = 16
NW = NC * NS
EPW = E // NW
C = 40
NCHUNK = EPW // C
NFL = N // C
LANES = 16
G = D // LANES


def _agg_body(x_hbm, ee_hbm, idxr_hbm, out_hbm,
              idx_b, eb3, rb2, acc, se3, sg2, ss3, si4):
    c = lax.axis_index("c")
    s = lax.axis_index("s")
    wid = c * NS + s
    ebase = wid * EPW

    @plsc.parallel_loop(0, C, unroll=2)
    def zrow(r):
        for j in range(G):
            eb3[0, r, pl.ds(j * LANES, LANES)] = jnp.zeros((LANES,),
                                                           jnp.float32)
    for t in range((NFL + NS - 1) // NS):
        idx = s + NS * t

        @pl.when(idx < NFL)
        def _():
            pltpu.sync_copy(eb3.at[0], acc.at[pl.ds(idx * C, C)])
    plsc.subcore_barrier()

    def issue_idx(k, s4):
        pltpu.async_copy(idxr_hbm.at[wid, k], idx_b.at[s4], si4.at[s4])

    def wait_idx(k, s4):
        pltpu.make_async_copy(idxr_hbm.at[wid, k], idx_b.at[s4],
                              si4.at[s4]).wait()

    def issue_loads(k, s3, s2, s4):
        pltpu.async_copy(ee_hbm.at[pl.ds(ebase + k * C, C)], eb3.at[s3],
                         se3.at[s3])
        pltpu.async_copy(x_hbm.at[idx_b.at[s4, 0]], rb2.at[s2], sg2.at[s2])

    def wait_loads(k, s3, s2, s4):
        pltpu.make_async_copy(ee_hbm.at[pl.ds(ebase + k * C, C)], eb3.at[s3],
                              se3.at[s3]).wait()
        pltpu.make_async_copy(x_hbm.at[idx_b.at[s4, 0]], rb2.at[s2],
                              sg2.at[s2]).wait()

    def compute(s3, s2):
        @plsc.parallel_loop(0, C, unroll=4)
        def edge(e):
            for j in range(G):
                sl = pl.ds(j * LANES, LANES)
                eb3[s3, e, sl] = jnp.maximum(
                    rb2[s2, e, sl] + eb3[s3, e, sl], 0.0)

    def issue_scatter(k, s3, s4):
        pltpu.async_copy(eb3.at[s3], acc.at[idx_b.at[s4, 1]], ss3.at[s3],
                         add=True)

    def wait_scatter(k, s3, s4):
        pltpu.make_async_copy(eb3.at[s3], acc.at[idx_b.at[s4, 1]],
                              ss3.at[s3]).wait()

    def step(k, off):
        s3, s2, s4 = off % 3, off % 2, off % 4
        if not isinstance(k, int) or k >= 2:
            wait_scatter(k - 2, (off - 2) % 3, (off - 2) % 4)
        if isinstance(k, int) and k + 1 >= NCHUNK:
            pass
        else:
            wait_idx(k + 1, (off + 1) % 4)
            issue_loads(k + 1, (off + 1) % 3, (off + 1) % 2, (off + 1) % 4)
        if not isinstance(k, int) or k + 2 < NCHUNK:
            issue_idx(k + 2, (off + 2) % 4)
        wait_loads(k, s3, s2, s4)
        compute(s3, s2)
        issue_scatter(k, s3, s4)

    issue_idx(0, 0)
    issue_idx(1, 1)
    wait_idx(0, 0)
    issue_loads(0, 0, 0, 0)
    for k in range(12):
        step(k, k)

    nblk = (NCHUNK - 12 - 10) // 12

    def block(i, carry):
        kb = 12 * i
        for off in range(12):
            step(kb + off, off)
        return carry
    lax.fori_loop(1, 1 + nblk, block, 0)

    for k in range(NCHUNK - 10, NCHUNK):
        step(k, k % 12)
    wait_scatter(NCHUNK - 2, (NCHUNK - 2) % 3, (NCHUNK - 2) % 4)
    wait_scatter(NCHUNK - 1, (NCHUNK - 1) % 3, (NCHUNK - 1) % 4)

    plsc.subcore_barrier()
    for t in range((NFL + NS - 1) // NS):
        idx = s + NS * t

        @pl.when(idx < NFL)
        def _():
            off = idx * C
            pltpu.sync_copy(acc.at[pl.ds(off, C)],
                            out_hbm.at[c, pl.ds(off, C)])


_agg = pl.kernel(
    _agg_body,
    out_type=jax.ShapeDtypeStruct((NC, N, D), jnp.float32),
    mesh=plsc.VectorSubcoreMesh(core_axis_name="c", subcore_axis_name="s",
                                num_cores=NC, num_subcores=NS),
    scratch_types=[
        pltpu.VMEM((4, 2, C), jnp.int32),
        pltpu.VMEM((3, C, D), jnp.float32),
        pltpu.VMEM((2, C, D), jnp.float32),
        pltpu.VMEM_SHARED((N, D), jnp.float32),
        pltpu.SemaphoreType.DMA((3,)),
        pltpu.SemaphoreType.DMA((2,)),
        pltpu.SemaphoreType.DMA((3,)),
        pltpu.SemaphoreType.DMA((4,)),
    ],
)


BE = 4000


def _ee_body(ea_ref, we_ref, be_ref, out_ref):
    out_ref[...] = jnp.dot(ea_ref[...], we_ref[...],
                           preferred_element_type=jnp.float32) + be_ref[0]


_edge_proj = pl.pallas_call(
    _ee_body,
    grid=(E // BE,),
    in_specs=[
        pl.BlockSpec((BE, DE), lambda e: (e, 0)),
        pl.BlockSpec((DE, D), lambda e: (0, 0)),
        pl.BlockSpec((1, D), lambda e: (0, 0)),
    ],
    out_specs=pl.BlockSpec((BE, D), lambda e: (e, 0)),
    out_shape=jax.ShapeDtypeStruct((E, D), jnp.float32),
)


R = 2000


def _node_common(x_ref, a_ref, w1_ref, b1_ref, w2_ref, b2_ref, g_ref, bb_ref):
    x = x_ref[...]
    h = x + a_ref[0] + a_ref[1]
    t = jnp.maximum(
        jnp.dot(h, w1_ref[...], preferred_element_type=jnp.float32)
        + b1_ref[0], 0.0)
    t = jnp.dot(t, w2_ref[...], preferred_element_type=jnp.float32) + b2_ref[0]
    mu = jnp.mean(t, axis=-1, keepdims=True)
    var = jnp.mean((t - mu) ** 2, axis=-1, keepdims=True)
    t = (t - mu) / jnp.sqrt(var + 1e-5) * g_ref[0] + bb_ref[0]
    return jax.nn.gelu(t) + x


def _node_mid_body(x_ref, a_ref, w1_ref, b1_ref, w2_ref, b2_ref, g_ref,
                   bb_ref, out_ref):
    out_ref[...] = _node_common(x_ref, a_ref, w1_ref, b1_ref, w2_ref, b2_ref,
                                g_ref, bb_ref)


def _node_last_body(x_ref, a_ref, w1_ref, b1_ref, w2_ref, b2_ref, g_ref,
                    bb_ref, wo_ref, bo_ref, out_ref):
    y = _node_common(x_ref, a_ref, w1_ref, b1_ref, w2_ref, b2_ref,
                     g_ref, bb_ref)
    out_ref[...] = jnp.dot(y, wo_ref[...],
                           preferred_element_type=jnp.float32) + bo_ref[0]


_NODE_SPECS = [
    pl.BlockSpec((R, D), lambda i: (i, 0)),
    pl.BlockSpec((NC, R, D), lambda i: (0, i, 0)),
    pl.BlockSpec((D, D), lambda i: (0, 0)),
    pl.BlockSpec((1, D), lambda i: (0, 0)),
    pl.BlockSpec((D, D), lambda i: (0, 0)),
    pl.BlockSpec((1, D), lambda i: (0, 0)),
    pl.BlockSpec((1, D), lambda i: (0, 0)),
    pl.BlockSpec((1, D), lambda i: (0, 0)),
]

_node_mid = pl.pallas_call(
    _node_mid_body,
    grid=(N // R,),
    in_specs=_NODE_SPECS,
    out_specs=pl.BlockSpec((R, D), lambda i: (i, 0)),
    out_shape=jax.ShapeDtypeStruct((N, D), jnp.float32),
)

_node_last = pl.pallas_call(
    _node_last_body,
    grid=(N // R,),
    in_specs=_NODE_SPECS + [
        pl.BlockSpec((D, D), lambda i: (0, 0)),
        pl.BlockSpec((1, D), lambda i: (0, 0)),
    ],
    out_specs=pl.BlockSpec((R, D), lambda i: (i, 0)),
    out_shape=jax.ShapeDtypeStruct((N, D), jnp.float32),
)


def kernel(x, edge_index, edge_attr, We, be, W1, b1, W2, b2, ln_g, ln_b,
           Wout, bout):
    idxr = jnp.stack([edge_index[0].reshape(NW, NCHUNK, C),
                      edge_index[1].reshape(NW, NCHUNK, C)], axis=2)
    ee = [_edge_proj(edge_attr, We[i], be[i][None]) for i in range(L)]
    for i in range(L):
        agg2 = _agg(x, ee[i], idxr)
        args = (x, agg2, W1[i], b1[i][None], W2[i], b2[i][None],
                ln_g[i][None], ln_b[i][None])
        if i < L - 1:
            x = _node_mid(*args)
        else:
            x = _node_last(*args, Wout, bout[None])
    return x
